# baseline (speedup 1.0000x reference)
import functools

import jax
import jax.numpy as jnp
from jax import lax
from jax.experimental import pallas as pl
from jax.experimental.pallas import tpu as pltpu
from jax.experimental.pallas import tpu_sc as plsc

_H = 128
_T = 32
_BLK = 2048
_WIN = 128
_EPS = 1e-12

_INV2PI = 0.15915494309189535
_RC1 = 6.28125
_RC2 = 0.0019353071795864769
_S = (0.9999995999200256, -0.16666552635411797, 0.008332402988781087,
      -0.00019808633342696967, 2.6997146371026314e-06,
      -2.0362244894134877e-08)


def _sin_poly(x):
    n = jnp.floor(x * _INV2PI + 0.5)
    r = x - n * _RC1
    r = r - n * _RC2
    r2 = r * r
    p = _S[5]
    p = p * r2 + _S[4]
    p = p * r2 + _S[3]
    p = p * r2 + _S[2]
    p = p * r2 + _S[1]
    p = p * r2 + _S[0]
    return r * p


def _sc_word_gather(word_emb, ids2):
    n = ids2.shape[1]
    mesh = plsc.VectorSubcoreMesh(core_axis_name="core", subcore_axis_name="subcore")

    @functools.partial(
        pl.kernel,
        out_type=jax.ShapeDtypeStruct((n, _H), word_emb.dtype),
        mesh=mesh,
    )
    def gather_kernel(tab_hbm, idx_hbm, out_hbm):
        def body(idx_vmem, out_vmem):
            pltpu.sync_copy(tab_hbm.at[idx_vmem.at[0]], out_vmem)

        pltpu.emit_pipeline(
            body,
            grid=(n // _WIN,),
            in_specs=[pl.BlockSpec((1, _WIN), index_map=lambda i: (0, i))],
            out_specs=[pl.BlockSpec((_WIN, _H), index_map=lambda i: (i, 0))],
            core_axis_name=("core", "subcore"),
            dimension_semantics=(pltpu.PARALLEL,),
        )(idx_hbm, out_hbm)

    return gather_kernel(word_emb, ids2)


def _tc_body(g_ref, scal_ref, iota_ref, w_ref, b_ref, twT_ref, tpT_ref,
             awT_ref, apT_ref, tte_ref, vse_ref, voe_ref, gamma_ref, beta_ref,
             out_ref):
    f32 = jnp.float32
    cdim = (((0,), (0,)), ((), ()))
    x = scal_ref[...]
    dt = x[0:1]
    age = x[1:2]
    ttv = x[2:3]
    vov = x[3:4]
    vsv = x[4:5]
    teT = _sin_poly(twT_ref[...] * dt + tpT_ref[...])
    aeT = _sin_poly(awT_ref[...] * age + apT_ref[...])
    w = w_ref[...]
    mm = jnp.dot(g_ref[...], w[0:_H], preferred_element_type=f32)
    mm = mm + lax.dot_general(teT, w[_H:_H + _T], cdim, preferred_element_type=f32)
    mm = mm + lax.dot_general(aeT, w[_H + _T:], cdim, preferred_element_type=f32)
    h = jnp.tanh(mm + b_ref[...])
    nt = tte_ref.shape[0]
    ns = vse_ref.shape[0]
    iota = iota_ref[...]
    oh_t = jnp.where(iota[0:nt] == ttv, 1.0, 0.0)
    oh_s = jnp.where(iota[0:ns] == vsv, 1.0, 0.0)
    oh_v = jnp.where(iota == vov, 1.0, 0.0)
    s = lax.dot_general(oh_t, tte_ref[...], cdim, preferred_element_type=f32)
    s = s + lax.dot_general(oh_s, vse_ref[...], cdim, preferred_element_type=f32)
    s = s + lax.dot_general(oh_v, voe_ref[...], cdim, preferred_element_type=f32)
    emb = h + s
    mu = jnp.mean(emb, axis=1, keepdims=True)
    d0 = emb - mu
    var = jnp.mean(d0 * d0, axis=1, keepdims=True)
    out_ref[...] = d0 * lax.rsqrt(var + _EPS) * gamma_ref[...] + beta_ref[...]


def _fused_tc(g, scal, iota_col, W, b2, twT, tpT, awT, apT,
              type_emb, visit_seg_emb, visit_order_emb, gamma2, beta2,
              n_total, blk_off, outbuf=None):
    csz = g.shape[0]
    grid = csz // _BLK
    tok = lambda i: (i, 0)
    out_tok = lambda i: (blk_off + i, 0)
    lane = lambda i: (0, i)
    rep = lambda i: (0, 0)
    in_specs = [
        pl.BlockSpec((_BLK, _H), tok),
        pl.BlockSpec((scal.shape[0], _BLK), lane),
        pl.BlockSpec((iota_col.shape[0], 1), rep),
        pl.BlockSpec((_H + 2 * _T, _H), rep),
        pl.BlockSpec((1, _H), rep),
        pl.BlockSpec((_T, 1), rep),
        pl.BlockSpec((_T, 1), rep),
        pl.BlockSpec((_T, 1), rep),
        pl.BlockSpec((_T, 1), rep),
        pl.BlockSpec((type_emb.shape[0], _H), rep),
        pl.BlockSpec((visit_seg_emb.shape[0], _H), rep),
        pl.BlockSpec((visit_order_emb.shape[0], _H), rep),
        pl.BlockSpec((1, _H), rep),
        pl.BlockSpec((1, _H), rep),
    ]
    args = [g, scal, iota_col, W, b2, twT, tpT, awT, apT,
            type_emb, visit_seg_emb, visit_order_emb, gamma2, beta2]
    body = _tc_body
    extra = {}
    if outbuf is not None:
        in_specs.append(pl.BlockSpec(memory_space=pl.ANY))
        args.append(outbuf)
        body = lambda *refs: _tc_body(*refs[:14], refs[15])
        extra = dict(input_output_aliases={14: 0})
    return pl.pallas_call(
        body,
        grid=(grid,),
        in_specs=in_specs,
        out_specs=pl.BlockSpec((_BLK, _H), out_tok),
        out_shape=jax.ShapeDtypeStruct((n_total, _H), jnp.float32),
        compiler_params=pltpu.CompilerParams(dimension_semantics=("arbitrary",)),
        **extra,
    )(*args)


def kernel(input_ids, token_type_ids_batch, time_stamps, ages, visit_orders,
           visit_segments, word_emb, type_emb, visit_order_emb, visit_seg_emb,
           time_w, time_phi, age_w, age_phi, W, b, gamma, beta):
    bsz, seq = input_ids.shape
    n = bsz * seq
    ids2 = input_ids.astype(jnp.int32).reshape(1, n)

    dt = time_stamps - jnp.concatenate(
        [time_stamps[:, :1], time_stamps[:, :-1]], axis=1)
    scal = jnp.stack([
        dt.reshape(n),
        ages.reshape(n),
        token_type_ids_batch.astype(jnp.float32).reshape(n),
        visit_orders.astype(jnp.float32).reshape(n),
        visit_segments.astype(jnp.float32).reshape(n),
    ], axis=0)
    nv = visit_order_emb.shape[0]
    iota_col = jnp.arange(nv, dtype=jnp.float32).reshape(nv, 1)
    b2 = b.reshape(1, _H)
    gamma2 = gamma.reshape(1, _H)
    beta2 = beta.reshape(1, _H)
    twT = time_w.reshape(_T, 1)
    tpT = time_phi.reshape(_T, 1)
    awT = age_w.reshape(_T, 1)
    apT = age_phi.reshape(_T, 1)

    nchunks = 2
    csz = n // nchunks
    gs = [_sc_word_gather(word_emb, ids2[:, k * csz:(k + 1) * csz])
          for k in range(nchunks)]
    out = None
    for k in range(nchunks):
        sl = slice(k * csz, (k + 1) * csz)
        out = _fused_tc(gs[k], scal[:, sl], iota_col, W, b2, twT, tpT,
                        awT, apT, type_emb, visit_seg_emb, visit_order_emb,
                        gamma2, beta2, n, k * (csz // _BLK), outbuf=out)
    return out.reshape(bsz, seq, _H)


# speedup vs baseline: 1.0187x; 1.0187x over previous
import functools

import jax
import jax.numpy as jnp
from jax import lax
from jax.experimental import pallas as pl
from jax.experimental.pallas import tpu as pltpu
from jax.experimental.pallas import tpu_sc as plsc

_H = 128
_T = 32
_BLK = 4096
_WIN = 128
_EPS = 1e-12

_INV2PI = 0.15915494309189535
_RC1 = 6.28125
_RC2 = 0.0019353071795864769
_S = (0.9999995999200256, -0.16666552635411797, 0.008332402988781087,
      -0.00019808633342696967, 2.6997146371026314e-06,
      -2.0362244894134877e-08)


def _sin_poly(x):
    n = jnp.floor(x * _INV2PI + 0.5)
    r = x - n * _RC1
    r = r - n * _RC2
    r2 = r * r
    p = _S[5]
    p = p * r2 + _S[4]
    p = p * r2 + _S[3]
    p = p * r2 + _S[2]
    p = p * r2 + _S[1]
    p = p * r2 + _S[0]
    return r * p


def _sc_word_gather(word_emb, ids2):
    n = ids2.shape[1]
    mesh = plsc.VectorSubcoreMesh(core_axis_name="core", subcore_axis_name="subcore")

    @functools.partial(
        pl.kernel,
        out_type=jax.ShapeDtypeStruct((n, _H), word_emb.dtype),
        mesh=mesh,
    )
    def gather_kernel(tab_hbm, idx_hbm, out_hbm):
        def body(idx_vmem, out_vmem):
            pltpu.sync_copy(tab_hbm.at[idx_vmem.at[0]], out_vmem)

        pltpu.emit_pipeline(
            body,
            grid=(n // _WIN,),
            in_specs=[pl.BlockSpec((1, _WIN), index_map=lambda i: (0, i))],
            out_specs=[pl.BlockSpec((_WIN, _H), index_map=lambda i: (i, 0))],
            core_axis_name=("core", "subcore"),
            dimension_semantics=(pltpu.PARALLEL,),
        )(idx_hbm, out_hbm)

    return gather_kernel(word_emb, ids2)


def _tc_body(g_ref, scal_ref, iota_ref, w_ref, b_ref, twT_ref, tpT_ref,
             awT_ref, apT_ref, tte_ref, vse_ref, voe_ref, gamma_ref, beta_ref,
             out_ref):
    f32 = jnp.float32
    cdim = (((0,), (0,)), ((), ()))
    x = scal_ref[...]
    dt = x[0:1]
    age = x[1:2]
    ttv = x[2:3]
    vov = x[3:4]
    vsv = x[4:5]
    teT = _sin_poly(twT_ref[...] * dt + tpT_ref[...])
    aeT = _sin_poly(awT_ref[...] * age + apT_ref[...])
    w = w_ref[...]
    mm = jnp.dot(g_ref[...], w[0:_H], preferred_element_type=f32)
    mm = mm + lax.dot_general(teT, w[_H:_H + _T], cdim, preferred_element_type=f32)
    mm = mm + lax.dot_general(aeT, w[_H + _T:], cdim, preferred_element_type=f32)
    h = jnp.tanh(mm + b_ref[...])
    nt = tte_ref.shape[0]
    ns = vse_ref.shape[0]
    iota = iota_ref[...]
    oh_t = jnp.where(iota[0:nt] == ttv, 1.0, 0.0)
    oh_s = jnp.where(iota[0:ns] == vsv, 1.0, 0.0)
    oh_v = jnp.where(iota == vov, 1.0, 0.0)
    s = lax.dot_general(oh_t, tte_ref[...], cdim, preferred_element_type=f32)
    s = s + lax.dot_general(oh_s, vse_ref[...], cdim, preferred_element_type=f32)
    s = s + lax.dot_general(oh_v, voe_ref[...], cdim, preferred_element_type=f32)
    emb = h + s
    mu = jnp.mean(emb, axis=1, keepdims=True)
    d0 = emb - mu
    var = jnp.mean(d0 * d0, axis=1, keepdims=True)
    out_ref[...] = d0 * lax.rsqrt(var + _EPS) * gamma_ref[...] + beta_ref[...]


def _fused_tc(g, scal, iota_col, W, b2, twT, tpT, awT, apT,
              type_emb, visit_seg_emb, visit_order_emb, gamma2, beta2,
              n_total, blk_off, outbuf=None):
    csz = g.shape[0]
    grid = csz // _BLK
    tok = lambda i: (i, 0)
    out_tok = lambda i: (blk_off + i, 0)
    lane = lambda i: (0, i)
    rep = lambda i: (0, 0)
    in_specs = [
        pl.BlockSpec((_BLK, _H), tok),
        pl.BlockSpec((scal.shape[0], _BLK), lane),
        pl.BlockSpec((iota_col.shape[0], 1), rep),
        pl.BlockSpec((_H + 2 * _T, _H), rep),
        pl.BlockSpec((1, _H), rep),
        pl.BlockSpec((_T, 1), rep),
        pl.BlockSpec((_T, 1), rep),
        pl.BlockSpec((_T, 1), rep),
        pl.BlockSpec((_T, 1), rep),
        pl.BlockSpec((type_emb.shape[0], _H), rep),
        pl.BlockSpec((visit_seg_emb.shape[0], _H), rep),
        pl.BlockSpec((visit_order_emb.shape[0], _H), rep),
        pl.BlockSpec((1, _H), rep),
        pl.BlockSpec((1, _H), rep),
    ]
    args = [g, scal, iota_col, W, b2, twT, tpT, awT, apT,
            type_emb, visit_seg_emb, visit_order_emb, gamma2, beta2]
    body = _tc_body
    extra = {}
    if outbuf is not None:
        in_specs.append(pl.BlockSpec(memory_space=pl.ANY))
        args.append(outbuf)
        body = lambda *refs: _tc_body(*refs[:14], refs[15])
        extra = dict(input_output_aliases={14: 0})
    return pl.pallas_call(
        body,
        grid=(grid,),
        in_specs=in_specs,
        out_specs=pl.BlockSpec((_BLK, _H), out_tok),
        out_shape=jax.ShapeDtypeStruct((n_total, _H), jnp.float32),
        compiler_params=pltpu.CompilerParams(dimension_semantics=("arbitrary",)),
        **extra,
    )(*args)


def kernel(input_ids, token_type_ids_batch, time_stamps, ages, visit_orders,
           visit_segments, word_emb, type_emb, visit_order_emb, visit_seg_emb,
           time_w, time_phi, age_w, age_phi, W, b, gamma, beta):
    bsz, seq = input_ids.shape
    n = bsz * seq
    ids2 = input_ids.astype(jnp.int32).reshape(1, n)

    dt = time_stamps - jnp.concatenate(
        [time_stamps[:, :1], time_stamps[:, :-1]], axis=1)
    scal = jnp.stack([
        dt.reshape(n),
        ages.reshape(n),
        token_type_ids_batch.astype(jnp.float32).reshape(n),
        visit_orders.astype(jnp.float32).reshape(n),
        visit_segments.astype(jnp.float32).reshape(n),
    ], axis=0)
    nv = visit_order_emb.shape[0]
    iota_col = jnp.arange(nv, dtype=jnp.float32).reshape(nv, 1)
    b2 = b.reshape(1, _H)
    gamma2 = gamma.reshape(1, _H)
    beta2 = beta.reshape(1, _H)
    twT = time_w.reshape(_T, 1)
    tpT = time_phi.reshape(_T, 1)
    awT = age_w.reshape(_T, 1)
    apT = age_phi.reshape(_T, 1)

    nchunks = 5
    csz = n // nchunks
    gs = [_sc_word_gather(word_emb, ids2[:, k * csz:(k + 1) * csz])
          for k in range(nchunks)]
    out = None
    for k in range(nchunks):
        sl = slice(k * csz, (k + 1) * csz)
        out = _fused_tc(gs[k], scal[:, sl], iota_col, W, b2, twT, tpT,
                        awT, apT, type_emb, visit_seg_emb, visit_order_emb,
                        gamma2, beta2, n, k * (csz // _BLK), outbuf=out)
    return out.reshape(bsz, seq, _H)


# speedup vs baseline: 1.0259x; 1.0070x over previous
import functools

import jax
import jax.numpy as jnp
from jax import lax
from jax.experimental import pallas as pl
from jax.experimental.pallas import tpu as pltpu
from jax.experimental.pallas import tpu_sc as plsc

_H = 128
_T = 32
_BLK = 4096
_WIN = 256
_EPS = 1e-12

_INV2PI = 0.15915494309189535
_RC1 = 6.28125
_RC2 = 0.0019353071795864769
_S = (0.9999995999200256, -0.16666552635411797, 0.008332402988781087,
      -0.00019808633342696967, 2.6997146371026314e-06,
      -2.0362244894134877e-08)


def _sin_poly(x):
    n = jnp.floor(x * _INV2PI + 0.5)
    r = x - n * _RC1
    r = r - n * _RC2
    r2 = r * r
    p = _S[5]
    p = p * r2 + _S[4]
    p = p * r2 + _S[3]
    p = p * r2 + _S[2]
    p = p * r2 + _S[1]
    p = p * r2 + _S[0]
    return r * p


def _sc_word_gather(word_emb, ids2):
    n = ids2.shape[1]
    mesh = plsc.VectorSubcoreMesh(core_axis_name="core", subcore_axis_name="subcore")

    @functools.partial(
        pl.kernel,
        out_type=jax.ShapeDtypeStruct((n, _H), word_emb.dtype),
        mesh=mesh,
    )
    def gather_kernel(tab_hbm, idx_hbm, out_hbm):
        def body(idx_vmem, out_vmem):
            pltpu.sync_copy(tab_hbm.at[idx_vmem.at[0]], out_vmem)

        pltpu.emit_pipeline(
            body,
            grid=(n // _WIN,),
            in_specs=[pl.BlockSpec((1, _WIN), index_map=lambda i: (0, i))],
            out_specs=[pl.BlockSpec((_WIN, _H), index_map=lambda i: (i, 0))],
            core_axis_name=("core", "subcore"),
            dimension_semantics=(pltpu.PARALLEL,),
        )(idx_hbm, out_hbm)

    return gather_kernel(word_emb, ids2)


def _tc_body(g_ref, scal_ref, iota_ref, w_ref, b_ref, twT_ref, tpT_ref,
             awT_ref, apT_ref, tte_ref, vse_ref, voe_ref, gamma_ref, beta_ref,
             out_ref):
    f32 = jnp.float32
    cdim = (((0,), (0,)), ((), ()))
    x = scal_ref[...]
    dt = x[0:1]
    age = x[1:2]
    ttv = x[2:3]
    vov = x[3:4]
    vsv = x[4:5]
    teT = _sin_poly(twT_ref[...] * dt + tpT_ref[...])
    aeT = _sin_poly(awT_ref[...] * age + apT_ref[...])
    w = w_ref[...]
    mm = jnp.dot(g_ref[...], w[0:_H], preferred_element_type=f32)
    mm = mm + lax.dot_general(teT, w[_H:_H + _T], cdim, preferred_element_type=f32)
    mm = mm + lax.dot_general(aeT, w[_H + _T:], cdim, preferred_element_type=f32)
    h = jnp.tanh(mm + b_ref[...])
    nt = tte_ref.shape[0]
    ns = vse_ref.shape[0]
    iota = iota_ref[...]
    oh_t = jnp.where(iota[0:nt] == ttv, 1.0, 0.0)
    oh_s = jnp.where(iota[0:ns] == vsv, 1.0, 0.0)
    oh_v = jnp.where(iota == vov, 1.0, 0.0)
    s = lax.dot_general(oh_t, tte_ref[...], cdim, preferred_element_type=f32)
    s = s + lax.dot_general(oh_s, vse_ref[...], cdim, preferred_element_type=f32)
    s = s + lax.dot_general(oh_v, voe_ref[...], cdim, preferred_element_type=f32)
    emb = h + s
    mu = jnp.mean(emb, axis=1, keepdims=True)
    d0 = emb - mu
    var = jnp.mean(d0 * d0, axis=1, keepdims=True)
    out_ref[...] = d0 * lax.rsqrt(var + _EPS) * gamma_ref[...] + beta_ref[...]


def _fused_tc(g, scal, iota_col, W, b2, twT, tpT, awT, apT,
              type_emb, visit_seg_emb, visit_order_emb, gamma2, beta2,
              n_total, blk_off, outbuf=None):
    csz = g.shape[0]
    grid = csz // _BLK
    tok = lambda i: (i, 0)
    out_tok = lambda i: (blk_off + i, 0)
    lane = lambda i: (0, i)
    rep = lambda i: (0, 0)
    in_specs = [
        pl.BlockSpec((_BLK, _H), tok),
        pl.BlockSpec((scal.shape[0], _BLK), lane),
        pl.BlockSpec((iota_col.shape[0], 1), rep),
        pl.BlockSpec((_H + 2 * _T, _H), rep),
        pl.BlockSpec((1, _H), rep),
        pl.BlockSpec((_T, 1), rep),
        pl.BlockSpec((_T, 1), rep),
        pl.BlockSpec((_T, 1), rep),
        pl.BlockSpec((_T, 1), rep),
        pl.BlockSpec((type_emb.shape[0], _H), rep),
        pl.BlockSpec((visit_seg_emb.shape[0], _H), rep),
        pl.BlockSpec((visit_order_emb.shape[0], _H), rep),
        pl.BlockSpec((1, _H), rep),
        pl.BlockSpec((1, _H), rep),
    ]
    args = [g, scal, iota_col, W, b2, twT, tpT, awT, apT,
            type_emb, visit_seg_emb, visit_order_emb, gamma2, beta2]
    body = _tc_body
    extra = {}
    if outbuf is not None:
        in_specs.append(pl.BlockSpec(memory_space=pl.ANY))
        args.append(outbuf)
        body = lambda *refs: _tc_body(*refs[:14], refs[15])
        extra = dict(input_output_aliases={14: 0})
    return pl.pallas_call(
        body,
        grid=(grid,),
        in_specs=in_specs,
        out_specs=pl.BlockSpec((_BLK, _H), out_tok),
        out_shape=jax.ShapeDtypeStruct((n_total, _H), jnp.float32),
        compiler_params=pltpu.CompilerParams(dimension_semantics=("arbitrary",)),
        **extra,
    )(*args)


def kernel(input_ids, token_type_ids_batch, time_stamps, ages, visit_orders,
           visit_segments, word_emb, type_emb, visit_order_emb, visit_seg_emb,
           time_w, time_phi, age_w, age_phi, W, b, gamma, beta):
    bsz, seq = input_ids.shape
    n = bsz * seq
    ids2 = input_ids.astype(jnp.int32).reshape(1, n)

    dt = time_stamps - jnp.concatenate(
        [time_stamps[:, :1], time_stamps[:, :-1]], axis=1)
    scal = jnp.stack([
        dt.reshape(n),
        ages.reshape(n),
        token_type_ids_batch.astype(jnp.float32).reshape(n),
        visit_orders.astype(jnp.float32).reshape(n),
        visit_segments.astype(jnp.float32).reshape(n),
    ], axis=0)
    nv = visit_order_emb.shape[0]
    iota_col = jnp.arange(nv, dtype=jnp.float32).reshape(nv, 1)
    b2 = b.reshape(1, _H)
    gamma2 = gamma.reshape(1, _H)
    beta2 = beta.reshape(1, _H)
    twT = time_w.reshape(_T, 1)
    tpT = time_phi.reshape(_T, 1)
    awT = age_w.reshape(_T, 1)
    apT = age_phi.reshape(_T, 1)

    nchunks = 5
    csz = n // nchunks
    gs = [_sc_word_gather(word_emb, ids2[:, k * csz:(k + 1) * csz])
          for k in range(nchunks)]
    out = None
    for k in range(nchunks):
        sl = slice(k * csz, (k + 1) * csz)
        out = _fused_tc(gs[k], scal[:, sl], iota_col, W, b2, twT, tpT,
                        awT, apT, type_emb, visit_seg_emb, visit_order_emb,
                        gamma2, beta2, n, k * (csz // _BLK), outbuf=out)
    return out.reshape(bsz, seq, _H)
